# baseline (speedup 1.0000x reference)
import functools

import numpy as np
import jax
import jax.numpy as jnp
from jax import lax
from jax.experimental import pallas as pl
from jax.experimental.pallas import tpu as pltpu
from jax.experimental.pallas import tpu_sc as plsc

_SEQ = 8192
_D = 512
_K = _D // 2
_LANES = 16
_NC = 2
_NS = 16
_NW = _NC * _NS
_BPW = _SEQ // _NW
_C = 16
_NCH = _BPW // _C
_NBUF = 8
_PF = 4


def _pe_tables_np():
    om = 1.0 / np.power(10000.0, 2.0 * np.arange(_K, dtype=np.float64) / _D)
    w_ = np.arange(_NW, dtype=np.float64)
    a_ = np.arange(16, dtype=np.float64)
    b_ = np.arange(16, dtype=np.float64)

    def inter(x, y):
        out = np.empty(x.shape[:-1] + (_D,), np.float64)
        out[..., 0::2] = x
        out[..., 1::2] = y
        return out

    ang_wa = (256.0 * w_[:, None, None] + 16.0 * a_[None, :, None]) * om
    sw, cw = np.sin(ang_wa), np.cos(ang_wa)
    wa = np.stack([inter(sw, cw), inter(cw, -sw)], axis=2).astype(np.float32)
    ang_b = b_[:, None] * om[None, :]
    sb, cb = np.sin(ang_b), np.cos(ang_b)
    bt = np.stack([inter(cb, cb), inter(sb, sb)], axis=1).astype(np.float32)
    return wa, bt


_WA_NP, _BT_NP = _pe_tables_np()


def _body(x_hbm, table_hbm, wa_hbm, bt_hbm, out_hbm,
          idx_v, wa_v, bt_v, rows_v, sem_g, sem_o, sem_t):
    wid = lax.axis_index("s") * _NC + lax.axis_index("c")
    base = wid * _BPW
    pltpu.sync_copy(x_hbm.at[pl.ds(base, _BPW)], idx_v)
    t_wa = pltpu.async_copy(wa_hbm.at[wid], wa_v, sem_t)
    t_bt = pltpu.async_copy(bt_hbm, bt_v, sem_t)

    def start_gather(ch):
        return pltpu.async_copy(
            table_hbm.at[idx_v.at[pl.ds(ch * _C, _C)]], rows_v.at[ch % _NBUF],
            sem_g.at[ch % _NBUF])

    g, o = {}, {}
    for ch in range(_PF):
        g[ch] = start_gather(ch)
    t_wa.wait()
    t_bt.wait()

    for ch in range(_NCH):
        b = ch % _NBUF
        nxt = ch + _PF
        if nxt < _NCH:
            if nxt - _NBUF >= 0:
                o[nxt - _NBUF].wait()
            g[nxt] = start_gather(nxt)
        g[ch].wait()

        def g_body(gg, _):
            goff = gg * _LANES
            bd = [(bt_v[bb, 0, pl.ds(goff, _LANES)],
                   bt_v[bb, 1, pl.ds(goff, _LANES)]) for bb in range(16)]
            for a_off in range(_C // 16):
                a = (_C // 16) * ch + a_off
                av = wa_v[a, 0, pl.ds(goff, _LANES)]
                cv = wa_v[a, 1, pl.ds(goff, _LANES)]
                for bb in range(16):
                    bv, dv = bd[bb]
                    plsc.addupdate(
                        rows_v.at[b, a_off * 16 + bb, pl.ds(goff, _LANES)],
                        av * bv + cv * dv,
                    )
            return 0

        lax.fori_loop(0, _D // _LANES, g_body, 0)

        o[ch] = pltpu.async_copy(
            rows_v.at[b], out_hbm.at[pl.ds(base + ch * _C, _C)], sem_o.at[b])

    for ch in range(_NCH - min(_NBUF, _NCH), _NCH):
        o[ch].wait()


_sc_kernel = functools.partial(
    pl.kernel,
    out_type=jax.ShapeDtypeStruct((_SEQ, _D), jnp.float32),
    mesh=plsc.VectorSubcoreMesh(core_axis_name="c", subcore_axis_name="s"),
    scratch_types=[
        pltpu.VMEM((_BPW,), jnp.int32),
        pltpu.VMEM((16, 2, _D), jnp.float32),
        pltpu.VMEM((16, 2, _D), jnp.float32),
        pltpu.VMEM((_NBUF, _C, _D), jnp.float32),
        pltpu.SemaphoreType.DMA((_NBUF,)),
        pltpu.SemaphoreType.DMA((_NBUF,)),
        pltpu.SemaphoreType.DMA,
    ],
)(_body)


def kernel(x, table):
    wa = jnp.asarray(_WA_NP)
    bt = jnp.asarray(_BT_NP)
    return _sc_kernel(x.astype(jnp.int32), table, wa, bt)


# speedup vs baseline: 1.1129x; 1.1129x over previous
import functools

import numpy as np
import jax
import jax.numpy as jnp
from jax import lax
from jax.experimental import pallas as pl
from jax.experimental.pallas import tpu as pltpu
from jax.experimental.pallas import tpu_sc as plsc

_SEQ = 8192
_D = 512
_K = _D // 2
_LANES = 16
_NC = 2
_NS = 16
_NW = _NC * _NS
_BPW = _SEQ // _NW
_C = 32
_NCH = _BPW // _C
_NBUF = 4
_PF = 2


def _pe_tables_np():
    om = 1.0 / np.power(10000.0, 2.0 * np.arange(_K, dtype=np.float64) / _D)
    w_ = np.arange(_NW, dtype=np.float64)
    a_ = np.arange(16, dtype=np.float64)
    b_ = np.arange(16, dtype=np.float64)

    def inter(x, y):
        out = np.empty(x.shape[:-1] + (_D,), np.float64)
        out[..., 0::2] = x
        out[..., 1::2] = y
        return out

    ang_wa = (256.0 * w_[:, None, None] + 16.0 * a_[None, :, None]) * om
    sw, cw = np.sin(ang_wa), np.cos(ang_wa)
    wa = np.stack([inter(sw, cw), inter(cw, -sw)], axis=2).astype(np.float32)
    ang_b = b_[:, None] * om[None, :]
    sb, cb = np.sin(ang_b), np.cos(ang_b)
    bt = np.stack([inter(cb, cb), inter(sb, sb)], axis=1).astype(np.float32)
    return wa, bt


_WA_NP, _BT_NP = _pe_tables_np()


def _body(x_hbm, table_hbm, wa_hbm, bt_hbm, out_hbm,
          idx_v, wa_v, bt_v, rows_v, sem_g, sem_o, sem_t):
    wid = lax.axis_index("s") * _NC + lax.axis_index("c")
    base = wid * _BPW
    pltpu.sync_copy(x_hbm.at[pl.ds(base, _BPW)], idx_v)
    t_wa = pltpu.async_copy(wa_hbm.at[wid], wa_v, sem_t)
    t_bt = pltpu.async_copy(bt_hbm, bt_v, sem_t)

    def start_gather(ch):
        return pltpu.async_copy(
            table_hbm.at[idx_v.at[pl.ds(ch * _C, _C)]], rows_v.at[ch % _NBUF],
            sem_g.at[ch % _NBUF])

    g, o = {}, {}
    for ch in range(_PF):
        g[ch] = start_gather(ch)
    t_wa.wait()
    t_bt.wait()

    for ch in range(_NCH):
        b = ch % _NBUF
        nxt = ch + _PF
        if nxt < _NCH:
            if nxt - _NBUF >= 0:
                o[nxt - _NBUF].wait()
            g[nxt] = start_gather(nxt)
        g[ch].wait()

        def g_body(gg, _):
            goff = gg * _LANES
            bd = [(bt_v[bb, 0, pl.ds(goff, _LANES)],
                   bt_v[bb, 1, pl.ds(goff, _LANES)]) for bb in range(16)]
            for a_off in range(_C // 16):
                a = (_C // 16) * ch + a_off
                av = wa_v[a, 0, pl.ds(goff, _LANES)]
                cv = wa_v[a, 1, pl.ds(goff, _LANES)]
                for bb in range(16):
                    bv, dv = bd[bb]
                    plsc.addupdate(
                        rows_v.at[b, a_off * 16 + bb, pl.ds(goff, _LANES)],
                        av * bv + cv * dv,
                    )
            return 0

        lax.fori_loop(0, _D // _LANES, g_body, 0)

        o[ch] = pltpu.async_copy(
            rows_v.at[b], out_hbm.at[pl.ds(base + ch * _C, _C)], sem_o.at[b])

    for ch in range(_NCH - min(_NBUF, _NCH), _NCH):
        o[ch].wait()


_sc_kernel = functools.partial(
    pl.kernel,
    out_type=jax.ShapeDtypeStruct((_SEQ, _D), jnp.float32),
    mesh=plsc.VectorSubcoreMesh(core_axis_name="c", subcore_axis_name="s"),
    scratch_types=[
        pltpu.VMEM((_BPW,), jnp.int32),
        pltpu.VMEM((16, 2, _D), jnp.float32),
        pltpu.VMEM((16, 2, _D), jnp.float32),
        pltpu.VMEM((_NBUF, _C, _D), jnp.float32),
        pltpu.SemaphoreType.DMA((_NBUF,)),
        pltpu.SemaphoreType.DMA((_NBUF,)),
        pltpu.SemaphoreType.DMA,
    ],
)(_body)


def kernel(x, table):
    wa = jnp.asarray(_WA_NP)
    bt = jnp.asarray(_BT_NP)
    return _sc_kernel(x.astype(jnp.int32), table, wa, bt)


# speedup vs baseline: 1.1167x; 1.0034x over previous
import functools

import numpy as np
import jax
import jax.numpy as jnp
from jax import lax
from jax.experimental import pallas as pl
from jax.experimental.pallas import tpu as pltpu
from jax.experimental.pallas import tpu_sc as plsc

_SEQ = 8192
_D = 512
_K = _D // 2
_LANES = 16
_NC = 2
_NS = 16
_NW = _NC * _NS
_BPW = _SEQ // _NW
_C = 32
_NCH = _BPW // _C
_NBUF = 5
_PF = 2


def _pe_tables_np():
    om = 1.0 / np.power(10000.0, 2.0 * np.arange(_K, dtype=np.float64) / _D)
    w_ = np.arange(_NW, dtype=np.float64)
    a_ = np.arange(16, dtype=np.float64)
    b_ = np.arange(16, dtype=np.float64)

    def inter(x, y):
        out = np.empty(x.shape[:-1] + (_D,), np.float64)
        out[..., 0::2] = x
        out[..., 1::2] = y
        return out

    ang_wa = (256.0 * w_[:, None, None] + 16.0 * a_[None, :, None]) * om
    sw, cw = np.sin(ang_wa), np.cos(ang_wa)
    wa = np.stack([inter(sw, cw), inter(cw, -sw)], axis=2).astype(np.float32)
    ang_b = b_[:, None] * om[None, :]
    sb, cb = np.sin(ang_b), np.cos(ang_b)
    bt = np.stack([inter(cb, cb), inter(sb, sb)], axis=1).astype(np.float32)
    return wa, bt


_WA_NP, _BT_NP = _pe_tables_np()


def _body(x_hbm, table_hbm, wa_hbm, bt_hbm, out_hbm,
          idx_v, wa_v, bt_v, rows_v, sem_g, sem_o, sem_t):
    wid = lax.axis_index("s") * _NC + lax.axis_index("c")
    base = wid * _BPW
    pltpu.sync_copy(x_hbm.at[pl.ds(base, _BPW)], idx_v)
    t_wa = pltpu.async_copy(wa_hbm.at[wid], wa_v, sem_t)
    t_bt = pltpu.async_copy(bt_hbm, bt_v, sem_t)

    def start_gather(ch):
        return pltpu.async_copy(
            table_hbm.at[idx_v.at[pl.ds(ch * _C, _C)]], rows_v.at[ch % _NBUF],
            sem_g.at[ch % _NBUF])

    g, o = {}, {}
    for ch in range(_PF):
        g[ch] = start_gather(ch)
    t_wa.wait()
    t_bt.wait()

    for ch in range(_NCH):
        b = ch % _NBUF
        nxt = ch + _PF
        if nxt < _NCH:
            if nxt - _NBUF >= 0:
                o[nxt - _NBUF].wait()
            g[nxt] = start_gather(nxt)
        g[ch].wait()

        def g_body(gg, _):
            goff = gg * _LANES
            bd = [(bt_v[bb, 0, pl.ds(goff, _LANES)],
                   bt_v[bb, 1, pl.ds(goff, _LANES)]) for bb in range(16)]
            for a_off in range(_C // 16):
                a = (_C // 16) * ch + a_off
                av = wa_v[a, 0, pl.ds(goff, _LANES)]
                cv = wa_v[a, 1, pl.ds(goff, _LANES)]
                for bb in range(16):
                    bv, dv = bd[bb]
                    plsc.addupdate(
                        rows_v.at[b, a_off * 16 + bb, pl.ds(goff, _LANES)],
                        av * bv + cv * dv,
                    )
            return 0

        lax.fori_loop(0, _D // _LANES, g_body, 0)

        o[ch] = pltpu.async_copy(
            rows_v.at[b], out_hbm.at[pl.ds(base + ch * _C, _C)], sem_o.at[b])

    for ch in range(_NCH - min(_NBUF, _NCH), _NCH):
        o[ch].wait()


_sc_kernel = functools.partial(
    pl.kernel,
    out_type=jax.ShapeDtypeStruct((_SEQ, _D), jnp.float32),
    mesh=plsc.VectorSubcoreMesh(core_axis_name="c", subcore_axis_name="s"),
    scratch_types=[
        pltpu.VMEM((_BPW,), jnp.int32),
        pltpu.VMEM((16, 2, _D), jnp.float32),
        pltpu.VMEM((16, 2, _D), jnp.float32),
        pltpu.VMEM((_NBUF, _C, _D), jnp.float32),
        pltpu.SemaphoreType.DMA((_NBUF,)),
        pltpu.SemaphoreType.DMA((_NBUF,)),
        pltpu.SemaphoreType.DMA,
    ],
)(_body)


def kernel(x, table):
    wa = jnp.asarray(_WA_NP)
    bt = jnp.asarray(_BT_NP)
    return _sc_kernel(x.astype(jnp.int32), table, wa, bt)


# speedup vs baseline: 1.1196x; 1.0026x over previous
import functools

import numpy as np
import jax
import jax.numpy as jnp
from jax import lax
from jax.experimental import pallas as pl
from jax.experimental.pallas import tpu as pltpu
from jax.experimental.pallas import tpu_sc as plsc

_SEQ = 8192
_D = 512
_K = _D // 2
_LANES = 16
_NC = 2
_NS = 16
_NW = _NC * _NS
_BPW = _SEQ // _NW
_C = 32
_NCH = _BPW // _C
_NBUF = 4
_PF = 2


def _pe_tables_np():
    om = 1.0 / np.power(10000.0, 2.0 * np.arange(_K, dtype=np.float64) / _D)
    w_ = np.arange(_NW, dtype=np.float64)
    a_ = np.arange(16, dtype=np.float64)
    b_ = np.arange(16, dtype=np.float64)

    def inter(x, y):
        out = np.empty(x.shape[:-1] + (_D,), np.float64)
        out[..., 0::2] = x
        out[..., 1::2] = y
        return out

    ang_wa = (256.0 * w_[:, None, None] + 16.0 * a_[None, :, None]) * om
    sw, cw = np.sin(ang_wa), np.cos(ang_wa)
    wa = np.stack([inter(sw, cw), inter(cw, -sw)], axis=2).astype(np.float32)
    ang_b = b_[:, None] * om[None, :]
    sb, cb = np.sin(ang_b), np.cos(ang_b)
    bt = np.stack([inter(cb, cb), inter(sb, sb)], axis=1).astype(np.float32)
    return wa, bt


_WA_NP, _BT_NP = _pe_tables_np()


def _body(x_hbm, table_hbm, wa_hbm, bt_hbm, out_hbm,
          idx_v, wa_v, bt_v, rows_v, sem_g, sem_o, sem_t):
    wid = lax.axis_index("s") * _NC + lax.axis_index("c")
    base = wid * _BPW
    pltpu.sync_copy(x_hbm.at[pl.ds(base, _BPW)], idx_v)
    t_wa = pltpu.async_copy(wa_hbm.at[wid], wa_v, sem_t)
    t_bt = pltpu.async_copy(bt_hbm, bt_v, sem_t)

    def start_gather(ch):
        return pltpu.async_copy(
            table_hbm.at[idx_v.at[pl.ds(ch * _C, _C)]], rows_v.at[ch % _NBUF],
            sem_g.at[ch % _NBUF])

    g, o = {}, {}
    for ch in range(_PF):
        g[ch] = start_gather(ch)
    t_wa.wait()
    t_bt.wait()

    for ch in range(_NCH):
        b = ch % _NBUF
        nxt = ch + _PF
        if nxt < _NCH:
            if nxt - _NBUF >= 0:
                o[nxt - _NBUF].wait()
            g[nxt] = start_gather(nxt)
        g[ch].wait()

        def g_body(gg, _):
            goff = gg * _LANES
            bd = [(bt_v[bb, 0, pl.ds(goff, _LANES)],
                   bt_v[bb, 1, pl.ds(goff, _LANES)]) for bb in range(16)]
            for a_off in range(_C // 16):
                a = (_C // 16) * ch + a_off
                av = wa_v[a, 0, pl.ds(goff, _LANES)]
                cv = wa_v[a, 1, pl.ds(goff, _LANES)]
                for bb in range(16):
                    bv, dv = bd[bb]
                    plsc.addupdate(
                        rows_v.at[b, a_off * 16 + bb, pl.ds(goff, _LANES)],
                        av * bv + cv * dv,
                    )
            return 0

        lax.fori_loop(0, _D // _LANES, g_body, 0)

        o[ch] = pltpu.async_copy(
            rows_v.at[b], out_hbm.at[pl.ds(base + ch * _C, _C)], sem_o.at[b])

    for ch in range(_NCH - min(_NBUF, _NCH), _NCH):
        o[ch].wait()


_sc_kernel = functools.partial(
    pl.kernel,
    out_type=jax.ShapeDtypeStruct((_SEQ, _D), jnp.float32),
    mesh=plsc.VectorSubcoreMesh(core_axis_name="c", subcore_axis_name="s"),
    scratch_types=[
        pltpu.VMEM((_BPW,), jnp.int32),
        pltpu.VMEM((16, 2, _D), jnp.float32),
        pltpu.VMEM((16, 2, _D), jnp.float32),
        pltpu.VMEM((_NBUF, _C, _D), jnp.float32),
        pltpu.SemaphoreType.DMA((_NBUF,)),
        pltpu.SemaphoreType.DMA((_NBUF,)),
        pltpu.SemaphoreType.DMA,
    ],
)(_body)


def kernel(x, table):
    wa = jnp.asarray(_WA_NP)
    bt = jnp.asarray(_BT_NP)
    return _sc_kernel(x.astype(jnp.int32), table, wa, bt)
